# baseline (speedup 1.0000x reference)
import functools

import jax
import jax.numpy as jnp
from jax import lax
from jax.experimental import pallas as pl
from jax.experimental.pallas import tpu as pltpu
from jax.experimental.pallas import tpu_sc as plsc


@functools.cache
def _make_gather(B: int, D: int, C: int):
    info = plsc.get_sparse_core_info()
    NC, NS = info.num_cores, info.num_subcores
    NW = NC * NS
    assert B % (NW * C) == 0
    b_per_w = B // NW
    n_chunks = b_per_w // C
    mesh = plsc.VectorSubcoreMesh(core_axis_name="c", subcore_axis_name="s")

    @functools.partial(
        pl.kernel,
        mesh=mesh,
        out_type=jax.ShapeDtypeStruct((B, D), jnp.float32),
        scratch_types=[
            pltpu.VMEM((C,), jnp.int32),
            pltpu.VMEM((C, D), jnp.float32),
            pltpu.SemaphoreType.DMA,
        ],
        compiler_params=pltpu.CompilerParams(use_tc_tiling_on_sc=False),
    )
    def k(table_hbm, idx_hbm, out_hbm, idx_v, rows_v, sem):
        wid = lax.axis_index("s") * NC + lax.axis_index("c")
        base = wid * b_per_w

        def body(i, carry):
            off = base + i * C
            pltpu.sync_copy(idx_hbm.at[pl.ds(off, C)], idx_v)
            pltpu.async_copy(table_hbm.at[idx_v], rows_v, sem).wait()
            pltpu.sync_copy(rows_v, out_hbm.at[pl.ds(off, C)])
            return carry

        lax.fori_loop(0, n_chunks, body, 0)

    return k


def kernel(x, weight):
    n, s = x.shape
    B = n * s
    D = weight.shape[1]
    xf = x.reshape(B)
    out = _make_gather(B, D, 128)(weight, xf)
    return out.reshape(n, s, D)


# speedup vs baseline: 1.1432x; 1.1432x over previous
import functools

import jax
import jax.numpy as jnp
from jax import lax
from jax.experimental import pallas as pl
from jax.experimental.pallas import tpu as pltpu
from jax.experimental.pallas import tpu_sc as plsc


@functools.cache
def _make_gather(B: int, D: int, C: int):
    info = plsc.get_sparse_core_info()
    NC, NS = info.num_cores, info.num_subcores
    NW = NC * NS
    assert B % (NW * C) == 0
    b_per_w = B // NW
    n_chunks = b_per_w // C
    mesh = plsc.VectorSubcoreMesh(core_axis_name="c", subcore_axis_name="s")

    @functools.partial(
        pl.kernel,
        mesh=mesh,
        out_type=jax.ShapeDtypeStruct((B, D), jnp.float32),
        scratch_types=[
            pltpu.VMEM((C,), jnp.int32),
            pltpu.VMEM((C, D), jnp.float32),
            pltpu.SemaphoreType.DMA,
        ],
        compiler_params=pltpu.CompilerParams(use_tc_tiling_on_sc=False),
    )
    def k(table_hbm, idx_hbm, out_hbm, idx_v, rows_v, sem):
        wid = lax.axis_index("s") * NC + lax.axis_index("c")
        base = wid * b_per_w

        def body(i, carry):
            off = base + i * C
            pltpu.sync_copy(idx_hbm.at[pl.ds(off, C)], idx_v)
            pltpu.async_copy(table_hbm.at[idx_v], rows_v, sem).wait()
            pltpu.sync_copy(rows_v, out_hbm.at[pl.ds(off, C)])
            return carry

        lax.fori_loop(0, n_chunks, body, 0)

    return k


def kernel(x, weight):
    n, s = x.shape
    B = n * s
    D = weight.shape[1]
    xf = x.reshape(B)
    out = _make_gather(B, D, 512)(weight, xf)
    return out.reshape(n, s, D)


# speedup vs baseline: 1.1887x; 1.0398x over previous
import functools

import jax
import jax.numpy as jnp
from jax import lax
from jax.experimental import pallas as pl
from jax.experimental.pallas import tpu as pltpu
from jax.experimental.pallas import tpu_sc as plsc


@functools.cache
def _make_gather(B: int, D: int, C: int, NBUF: int):
    info = plsc.get_sparse_core_info()
    NC, NS = info.num_cores, info.num_subcores
    NW = NC * NS
    assert B % (NW * C) == 0
    b_per_w = B // NW
    n_chunks = b_per_w // C
    assert n_chunks % NBUF == 0
    n_groups = n_chunks // NBUF
    mesh = plsc.VectorSubcoreMesh(core_axis_name="c", subcore_axis_name="s")

    @functools.partial(
        pl.kernel,
        mesh=mesh,
        out_type=jax.ShapeDtypeStruct((B, D), jnp.float32),
        scratch_types=[
            pltpu.VMEM((NBUF, C), jnp.int32),
            pltpu.VMEM((NBUF, C, D), jnp.float32),
        ] + [pltpu.SemaphoreType.DMA] * (3 * NBUF),
        compiler_params=pltpu.CompilerParams(use_tc_tiling_on_sc=False),
    )
    def k(table_hbm, idx_hbm, out_hbm, idx_v, rows_v, *sems):
        sem_i, sem_g, sem_o = (
            sems[:NBUF], sems[NBUF:2 * NBUF], sems[2 * NBUF:])
        wid = lax.axis_index("s") * NC + lax.axis_index("c")
        base = wid * b_per_w

        def start_idx(j, b):
            pltpu.async_copy(
                idx_hbm.at[pl.ds(base + j * C, C)], idx_v.at[b], sem_i[b])

        def start_gather(b):
            pltpu.make_async_copy(
                idx_hbm.at[pl.ds(0, C)], idx_v.at[b], sem_i[b]).wait()
            pltpu.async_copy(
                table_hbm.at[idx_v.at[b]], rows_v.at[b], sem_g[b])

        def start_out(j, b):
            pltpu.make_async_copy(
                table_hbm.at[idx_v.at[b]], rows_v.at[b], sem_g[b]).wait()
            pltpu.async_copy(
                rows_v.at[b], out_hbm.at[pl.ds(base + j * C, C)], sem_o[b])

        def wait_out(b):
            pltpu.make_async_copy(
                rows_v.at[b], out_hbm.at[pl.ds(0, C)], sem_o[b]).wait()

        for b in range(NBUF):
            start_idx(b, b)
        for b in range(NBUF):
            start_gather(b)
        for b in range(NBUF):
            start_out(b, b)

        def body(g, carry):
            j0 = g * NBUF
            for b in range(NBUF):
                start_idx(j0 + b, b)
            for b in range(NBUF):
                wait_out(b)
                start_gather(b)
            for b in range(NBUF):
                start_out(j0 + b, b)
            return carry

        lax.fori_loop(1, n_groups, body, 0)

        for b in range(NBUF):
            wait_out(b)

    return k


def kernel(x, weight):
    n, s = x.shape
    B = n * s
    D = weight.shape[1]
    xf = x.reshape(B)
    out = _make_gather(B, D, 320, 5)(weight, xf)
    return out.reshape(n, s, D)


# speedup vs baseline: 1.4444x; 1.2151x over previous
import functools

import jax
import jax.numpy as jnp
from jax import lax
from jax.experimental import pallas as pl
from jax.experimental.pallas import tpu as pltpu
from jax.experimental.pallas import tpu_sc as plsc


@functools.cache
def _make_gather(B: int, V: int, C: int, NBUF: int):
    info = plsc.get_sparse_core_info()
    NC, NS = info.num_cores, info.num_subcores
    NW = NC * NS
    assert B % (NW * C) == 0
    b_per_w = B // NW
    n_chunks = b_per_w // C
    assert n_chunks % NBUF == 0
    n_groups = n_chunks // NBUF
    mesh = plsc.VectorSubcoreMesh(core_axis_name="c", subcore_axis_name="s")

    @functools.partial(
        pl.kernel,
        mesh=mesh,
        out_type=jax.ShapeDtypeStruct((B, 128), jnp.float32),
        scratch_types=(
            [pltpu.VMEM((C,), jnp.int32) for _ in range(NBUF)]
            + [pltpu.VMEM((C, 128), jnp.float32) for _ in range(NBUF)]
            + [pltpu.SemaphoreType.DMA] * (3 * NBUF)
        ),
        compiler_params=pltpu.CompilerParams(use_tc_tiling_on_sc=True),
    )
    def k(table_hbm, idx_hbm, out_hbm, *scratch):
        idx_v = scratch[:NBUF]
        rows_v = scratch[NBUF:2 * NBUF]
        sems = scratch[2 * NBUF:]
        sem_i, sem_g, sem_o = (
            sems[:NBUF], sems[NBUF:2 * NBUF], sems[2 * NBUF:])
        wid = lax.axis_index("s") * NC + lax.axis_index("c")
        base = wid * b_per_w

        def start_idx(j, b):
            pltpu.async_copy(
                idx_hbm.at[pl.ds(base + j * C, C)], idx_v[b], sem_i[b])

        def start_gather(b):
            pltpu.make_async_copy(
                idx_hbm.at[pl.ds(0, C)], idx_v[b], sem_i[b]).wait()
            pltpu.async_copy(
                table_hbm.at[idx_v[b]], rows_v[b], sem_g[b])

        def start_out(j, b):
            pltpu.make_async_copy(
                table_hbm.at[idx_v[b]], rows_v[b], sem_g[b]).wait()
            pltpu.async_copy(
                rows_v[b], out_hbm.at[pl.ds(base + j * C, C)], sem_o[b])

        def wait_out(b):
            pltpu.make_async_copy(
                rows_v[b], out_hbm.at[pl.ds(0, C)], sem_o[b]).wait()

        for b in range(NBUF):
            start_idx(b, b)
        for b in range(NBUF):
            start_gather(b)
        for b in range(NBUF):
            start_out(b, b)

        def body(g, carry):
            j0 = g * NBUF
            for b in range(NBUF):
                start_idx(j0 + b, b)
            for b in range(NBUF):
                wait_out(b)
                start_gather(b)
            for b in range(NBUF):
                start_out(j0 + b, b)
            return carry

        lax.fori_loop(1, n_groups, body, 0)

        for b in range(NBUF):
            wait_out(b)

    return k


def kernel(x, weight):
    n, s = x.shape
    B = n * s
    V, D = weight.shape
    wp = jnp.pad(weight, ((0, 0), (0, 128 - D)))
    xf = x.reshape(B)
    out = _make_gather(B, V, 200, 4)(wp, xf)
    return out.reshape(n, s, 128)[:, :, :D]
